# Initial kernel scaffold; baseline (speedup 1.0000x reference)
import functools

import jax
import jax.numpy as jnp
from jax import lax
from jax.experimental import pallas as pl
from jax.experimental.pallas import tpu as pltpu
from jax.experimental.pallas import tpu_sc as plsc

N = 10000
E = 320000
D_IN = 128
D_H = 256
EPS = 1e-5

N2 = 10240
E_PAD = 323584
NC = 2
NS = 16
CHUNK = 128
RPT = N2 // NS

_mesh = plsc.VectorSubcoreMesh(core_axis_name="c", subcore_axis_name="s")


def _sc_agg0(x_hbm, src_hbm, dst_hbm, zrows_hbm, zvec_hbm,
             agg_out, cnt_out,
             src_v, dst_v, rows_v, ones_v, acc_sh, cnt_sh):
    c = lax.axis_index("c")
    s = lax.axis_index("s")
    pltpu.sync_copy(zrows_hbm.at[pl.ds(s * RPT, RPT)],
                    acc_sh.at[pl.ds(s * RPT, RPT)])
    pltpu.sync_copy(zvec_hbm.at[pl.ds(s * RPT, RPT)],
                    cnt_sh.at[pl.ds(s * RPT, RPT)])
    for i in range(8):
        ones_v[pl.ds(i * 16, 16)] = jnp.full((16,), 1.0, jnp.float32)
    plsc.subcore_barrier()

    per_tile = E_PAD // (NC * NS)
    base = c * (E_PAD // NC) + s * per_tile
    nchunks = per_tile // CHUNK

    def chunk(j, carry):
        off = base + j * CHUNK
        pltpu.sync_copy(src_hbm.at[pl.ds(off, CHUNK)], src_v)
        pltpu.sync_copy(dst_hbm.at[pl.ds(off, CHUNK)], dst_v)
        pltpu.sync_copy(x_hbm.at[src_v], rows_v)
        pltpu.sync_copy(rows_v, acc_sh.at[dst_v], add=True)
        pltpu.sync_copy(ones_v, cnt_sh.at[dst_v], add=True)
        return carry

    lax.fori_loop(0, nchunks, chunk, 0)
    plsc.subcore_barrier()
    pltpu.sync_copy(acc_sh.at[pl.ds(s * RPT, RPT)],
                    agg_out.at[c, pl.ds(s * RPT, RPT)])
    pltpu.sync_copy(cnt_sh.at[pl.ds(s * RPT, RPT)],
                    cnt_out.at[c, pl.ds(s * RPT, RPT)])


def _sc_agg1(h_hbm, idx_hbm, dst_hbm, zrows_hbm,
             agg_out,
             src_v, dst_v, rows_v, acc_sh):
    c = lax.axis_index("c")
    s = lax.axis_index("s")
    pltpu.sync_copy(zrows_hbm.at[pl.ds(s * RPT, RPT)],
                    acc_sh.at[pl.ds(s * RPT, RPT)])
    plsc.subcore_barrier()

    per_tile = E_PAD // NS
    base_idx = c * E_PAD + s * per_tile
    base_dst = s * per_tile
    nchunks = per_tile // CHUNK

    def chunk(j, carry):
        pltpu.sync_copy(idx_hbm.at[pl.ds(base_idx + j * CHUNK, CHUNK)], src_v)
        pltpu.sync_copy(dst_hbm.at[pl.ds(base_dst + j * CHUNK, CHUNK)], dst_v)
        pltpu.sync_copy(h_hbm.at[src_v], rows_v)
        pltpu.sync_copy(rows_v, acc_sh.at[dst_v], add=True)
        return carry

    lax.fori_loop(0, nchunks, chunk, 0)
    plsc.subcore_barrier()
    pltpu.sync_copy(acc_sh.at[pl.ds(s * RPT, RPT)],
                    agg_out.at[c, pl.ds(s * RPT, RPT)])


def _sc_agg2(t_hbm, src_hbm, dst_hbm, zvec_hbm,
             agg_out,
             src_v, dst_v, vals_v, t_v, acc_sh):
    c = lax.axis_index("c")
    s = lax.axis_index("s")
    pltpu.sync_copy(t_hbm, t_v)
    pltpu.sync_copy(zvec_hbm.at[pl.ds(s * RPT, RPT)],
                    acc_sh.at[pl.ds(s * RPT, RPT)])
    plsc.subcore_barrier()

    per_tile = E_PAD // (NC * NS)
    base = c * (E_PAD // NC) + s * per_tile
    nchunks = per_tile // CHUNK

    def chunk(j, carry):
        off = base + j * CHUNK
        pltpu.sync_copy(src_hbm.at[pl.ds(off, CHUNK)], src_v)
        pltpu.sync_copy(dst_hbm.at[pl.ds(off, CHUNK)], dst_v)
        for i in range(8):
            idx16 = src_v[pl.ds(i * 16, 16)]
            vals_v[pl.ds(i * 16, 16)] = plsc.load_gather(t_v, [idx16])
        pltpu.sync_copy(vals_v, acc_sh.at[dst_v], add=True)
        return carry

    lax.fori_loop(0, nchunks, chunk, 0)
    plsc.subcore_barrier()
    pltpu.sync_copy(acc_sh.at[pl.ds(s * RPT, RPT)],
                    agg_out.at[c, pl.ds(s * RPT, RPT)])


_agg0 = pl.kernel(
    _sc_agg0,
    out_type=(jax.ShapeDtypeStruct((NC, N2, D_IN), jnp.float32),
              jax.ShapeDtypeStruct((NC, N2), jnp.float32)),
    mesh=_mesh,
    scratch_types=[
        pltpu.VMEM((CHUNK,), jnp.int32),
        pltpu.VMEM((CHUNK,), jnp.int32),
        pltpu.VMEM((CHUNK, D_IN), jnp.float32),
        pltpu.VMEM((CHUNK,), jnp.float32),
        pltpu.VMEM_SHARED((N2, D_IN), jnp.float32),
        pltpu.VMEM_SHARED((N2,), jnp.float32),
    ],
)

_agg1 = pl.kernel(
    _sc_agg1,
    out_type=jax.ShapeDtypeStruct((NC, N2, 128), jnp.float32),
    mesh=_mesh,
    scratch_types=[
        pltpu.VMEM((CHUNK,), jnp.int32),
        pltpu.VMEM((CHUNK,), jnp.int32),
        pltpu.VMEM((CHUNK, 128), jnp.float32),
        pltpu.VMEM_SHARED((N2, 128), jnp.float32),
    ],
)

_agg2 = pl.kernel(
    _sc_agg2,
    out_type=jax.ShapeDtypeStruct((NC, N2), jnp.float32),
    mesh=_mesh,
    scratch_types=[
        pltpu.VMEM((CHUNK,), jnp.int32),
        pltpu.VMEM((CHUNK,), jnp.int32),
        pltpu.VMEM((CHUNK,), jnp.float32),
        pltpu.VMEM((N2,), jnp.float32),
        pltpu.VMEM_SHARED((N2,), jnp.float32),
    ],
)

_BN_S = 1.0 / (1.0 + EPS) ** 0.5
_RB = 1024


def _tc_layer0(aggp, cntp, xp, wl0, wr0, b0, g0, be0, out):
    i = pl.program_id(0)
    agg = aggp[0] + aggp[1]
    cnt = cntp[0, pl.ds(i * _RB, _RB)] + cntp[1, pl.ds(i * _RB, _RB)]
    inv = 1.0 / jnp.maximum(cnt, 1.0)
    mean = agg * inv[:, None]
    z = (lax.dot_general(mean, wl0[...], (((1,), (1,)), ((), ())),
                         preferred_element_type=jnp.float32)
         + lax.dot_general(xp[...], wr0[...], (((1,), (1,)), ((), ())),
                           preferred_element_type=jnp.float32)
         + b0[0])
    h = z * (g0[0] * _BN_S) + be0[0]
    out[...] = jnp.maximum(h, 0.0)[None]


def _tc_layer1(aggp, cntp, h1p, wl1, wr1, b1, g1, be1, w2, out):
    i = pl.program_id(0)
    a = jnp.concatenate([aggp[0], aggp[1]], axis=1)
    hv = jnp.concatenate([h1p[0], h1p[1]], axis=1)
    cnt = cntp[0, pl.ds(i * _RB, _RB)] + cntp[1, pl.ds(i * _RB, _RB)]
    inv = 1.0 / jnp.maximum(cnt, 1.0)
    mean = a * inv[:, None]
    z = (lax.dot_general(mean, wl1[...], (((1,), (1,)), ((), ())),
                         preferred_element_type=jnp.float32)
         + lax.dot_general(hv, wr1[...], (((1,), (1,)), ((), ())),
                           preferred_element_type=jnp.float32)
         + b1[...])
    h2 = jnp.maximum(z * (g1[...] * _BN_S) + be1[...], 0.0)
    out[...] = jnp.dot(h2, w2[...], preferred_element_type=jnp.float32)


def _tc_final(tp, cntp, r2, b2, out):
    t = tp[...]
    tagg = t[0:1, :] + t[1:2, :]
    cnt = cntp[0:1, :] + cntp[1:2, :]
    inv = 1.0 / jnp.maximum(cnt, 1.0)
    val = tagg * inv + r2[...] + b2[0, 0]
    out[...] = jax.nn.sigmoid(val)


def kernel(x, edge_index, W_l0, b0, W_r0, gamma0, beta0,
           W_l1, b1, W_r1, gamma1, beta1, W_l2, b2, W_r2):
    f32 = jnp.float32
    src = edge_index[0]
    dst = edge_index[1]
    pad = E_PAD - E
    src_p = jnp.concatenate([src, jnp.zeros((pad,), jnp.int32)])
    dst_p = jnp.concatenate([dst, jnp.full((pad,), N, jnp.int32)])
    idx1 = jnp.concatenate([src_p, src_p + N2])
    xp = jnp.pad(x, ((0, N2 - N), (0, 0)))
    zrows = jnp.zeros((N2, 128), f32)
    zvec = jnp.zeros((N2,), f32)
    b0r = b0.reshape(2, 1, 128)
    g0r = gamma0.reshape(2, 1, 128)
    be0r = beta0.reshape(2, 1, 128)
    b1r = b1.reshape(1, D_H)
    g1r = gamma1.reshape(1, D_H)
    be1r = beta1.reshape(1, D_H)
    w2cat = jnp.concatenate([W_l2, W_r2], axis=0).T
    b2r = b2.reshape(1, 1)

    agg0p, cntp = _agg0(xp, src_p, dst_p, zrows, zvec)

    nblk = N2 // _RB
    h1s = pl.pallas_call(
        _tc_layer0,
        grid=(nblk, 2),
        in_specs=[
            pl.BlockSpec((NC, _RB, 128), lambda i, c: (0, i, 0)),
            pl.BlockSpec((NC, N2), lambda i, c: (0, 0)),
            pl.BlockSpec((_RB, 128), lambda i, c: (i, 0)),
            pl.BlockSpec((128, 128), lambda i, c: (c, 0)),
            pl.BlockSpec((128, 128), lambda i, c: (c, 0)),
            pl.BlockSpec((1, 1, 128), lambda i, c: (c, 0, 0)),
            pl.BlockSpec((1, 1, 128), lambda i, c: (c, 0, 0)),
            pl.BlockSpec((1, 1, 128), lambda i, c: (c, 0, 0)),
        ],
        out_specs=pl.BlockSpec((1, _RB, 128), lambda i, c: (c, i, 0)),
        out_shape=jax.ShapeDtypeStruct((2, N2, 128), f32),
    )(agg0p, cntp, xp, W_l0, W_r0, b0r, g0r, be0r)

    h1flat = h1s.reshape(2 * N2, 128)
    agg1p = _agg1(h1flat, idx1, dst_p, zrows)

    tr = pl.pallas_call(
        _tc_layer1,
        grid=(nblk,),
        in_specs=[
            pl.BlockSpec((NC, _RB, 128), lambda i: (0, i, 0)),
            pl.BlockSpec((NC, N2), lambda i: (0, 0)),
            pl.BlockSpec((NC, _RB, 128), lambda i: (0, i, 0)),
            pl.BlockSpec((D_H, D_H), lambda i: (0, 0)),
            pl.BlockSpec((D_H, D_H), lambda i: (0, 0)),
            pl.BlockSpec((1, D_H), lambda i: (0, 0)),
            pl.BlockSpec((1, D_H), lambda i: (0, 0)),
            pl.BlockSpec((1, D_H), lambda i: (0, 0)),
            pl.BlockSpec((D_H, 2), lambda i: (0, 0)),
        ],
        out_specs=pl.BlockSpec((_RB, 2), lambda i: (i, 0)),
        out_shape=jax.ShapeDtypeStruct((N2, 2), f32),
    )(agg1p, cntp, h1s, W_l1, W_r1, b1r, g1r, be1r, w2cat)

    t2 = tr[:, 0]
    r2row = tr[:, 1].reshape(1, N2)
    t2p = _agg2(t2, src_p, dst_p, zvec)

    outrow = pl.pallas_call(
        _tc_final,
        in_specs=[
            pl.BlockSpec((NC, N2), lambda: (0, 0)),
            pl.BlockSpec((NC, N2), lambda: (0, 0)),
            pl.BlockSpec((1, N2), lambda: (0, 0)),
            pl.BlockSpec((1, 1), lambda: (0, 0)),
        ],
        out_specs=pl.BlockSpec((1, N2), lambda: (0, 0)),
        out_shape=jax.ShapeDtypeStruct((1, N2), f32),
    )(t2p, cntp, r2row, b2r)

    return outrow[0, :N].reshape(N, 1)


# speedup vs baseline: 5.3497x; 5.3497x over previous
import functools

import jax
import jax.numpy as jnp
from jax import lax
from jax.experimental import pallas as pl
from jax.experimental.pallas import tpu as pltpu
from jax.experimental.pallas import tpu_sc as plsc

N = 10000
E = 320000
D_IN = 128
D_H = 256
EPS = 1e-5

N2 = 10240
E_PAD = 323584
NC = 2
NS = 16
CHUNK = 128
RPT = N2 // NS

_mesh = plsc.VectorSubcoreMesh(core_axis_name="c", subcore_axis_name="s")


def _sc_agg0(x_hbm, src_hbm, dst_hbm, zrows_hbm, zvec_hbm,
             agg_out, cnt_out,
             src_v, dst_v, rows_v, ones_v, acc_sh, cnt_sh):
    c = lax.axis_index("c")
    s = lax.axis_index("s")
    pltpu.sync_copy(zrows_hbm.at[pl.ds(s * RPT, RPT)],
                    acc_sh.at[pl.ds(s * RPT, RPT)])
    pltpu.sync_copy(zvec_hbm.at[pl.ds(s * RPT, RPT)],
                    cnt_sh.at[pl.ds(s * RPT, RPT)])
    for i in range(8):
        ones_v[pl.ds(i * 16, 16)] = jnp.full((16,), 1.0, jnp.float32)
    plsc.subcore_barrier()

    per_tile = E_PAD // (NC * NS)
    base = c * (E_PAD // NC) + s * per_tile
    nchunks = per_tile // CHUNK

    def chunk(j, carry):
        off = base + j * CHUNK
        pltpu.sync_copy(src_hbm.at[pl.ds(off, CHUNK)], src_v)
        pltpu.sync_copy(dst_hbm.at[pl.ds(off, CHUNK)], dst_v)
        pltpu.sync_copy(x_hbm.at[src_v], rows_v)
        pltpu.sync_copy(rows_v, acc_sh.at[dst_v], add=True)
        pltpu.sync_copy(ones_v, cnt_sh.at[dst_v], add=True)
        return carry

    lax.fori_loop(0, nchunks, chunk, 0)
    plsc.subcore_barrier()
    pltpu.sync_copy(acc_sh.at[pl.ds(s * RPT, RPT)],
                    agg_out.at[c, pl.ds(s * RPT, RPT)])
    pltpu.sync_copy(cnt_sh.at[pl.ds(s * RPT, RPT)],
                    cnt_out.at[c, pl.ds(s * RPT, RPT)])


def _sc_agg1(h_hbm, idx_hbm, dst_hbm, zrows_hbm,
             agg_out,
             src_v, dst_v, rows_v, acc_sh):
    c = lax.axis_index("c")
    s = lax.axis_index("s")
    pltpu.sync_copy(zrows_hbm.at[pl.ds(s * RPT, RPT)],
                    acc_sh.at[pl.ds(s * RPT, RPT)])
    plsc.subcore_barrier()

    per_tile = E_PAD // NS
    base_idx = c * E_PAD + s * per_tile
    base_dst = s * per_tile
    nchunks = per_tile // CHUNK

    def chunk(j, carry):
        pltpu.sync_copy(idx_hbm.at[pl.ds(base_idx + j * CHUNK, CHUNK)], src_v)
        pltpu.sync_copy(dst_hbm.at[pl.ds(base_dst + j * CHUNK, CHUNK)], dst_v)
        pltpu.sync_copy(h_hbm.at[src_v], rows_v)
        pltpu.sync_copy(rows_v, acc_sh.at[dst_v], add=True)
        return carry

    lax.fori_loop(0, nchunks, chunk, 0)
    plsc.subcore_barrier()
    pltpu.sync_copy(acc_sh.at[pl.ds(s * RPT, RPT)],
                    agg_out.at[c, pl.ds(s * RPT, RPT)])


def _sc_agg2(t_hbm, src_hbm, dst_hbm, zvec_hbm,
             agg_out,
             src_v, dst_v, vals_v, acc_sh):
    c = lax.axis_index("c")
    s = lax.axis_index("s")
    pltpu.sync_copy(zvec_hbm.at[pl.ds(s * RPT, RPT)],
                    acc_sh.at[pl.ds(s * RPT, RPT)])
    plsc.subcore_barrier()

    per_tile = E_PAD // (NC * NS)
    base = c * (E_PAD // NC) + s * per_tile
    nchunks = per_tile // CHUNK

    def chunk(j, carry):
        off = base + j * CHUNK
        pltpu.sync_copy(src_hbm.at[pl.ds(off, CHUNK)], src_v)
        pltpu.sync_copy(dst_hbm.at[pl.ds(off, CHUNK)], dst_v)
        pltpu.sync_copy(t_hbm.at[src_v], vals_v)
        pltpu.sync_copy(vals_v, acc_sh.at[dst_v], add=True)
        return carry

    lax.fori_loop(0, nchunks, chunk, 0)
    plsc.subcore_barrier()
    pltpu.sync_copy(acc_sh.at[pl.ds(s * RPT, RPT)],
                    agg_out.at[c, pl.ds(s * RPT, RPT)])


_agg0 = pl.kernel(
    _sc_agg0,
    out_type=(jax.ShapeDtypeStruct((NC, N2, D_IN), jnp.float32),
              jax.ShapeDtypeStruct((NC, N2), jnp.float32)),
    mesh=_mesh,
    scratch_types=[
        pltpu.VMEM((CHUNK,), jnp.int32),
        pltpu.VMEM((CHUNK,), jnp.int32),
        pltpu.VMEM((CHUNK, D_IN), jnp.float32),
        pltpu.VMEM((CHUNK,), jnp.float32),
        pltpu.VMEM_SHARED((N2, D_IN), jnp.float32),
        pltpu.VMEM_SHARED((N2,), jnp.float32),
    ],
)

_agg1 = pl.kernel(
    _sc_agg1,
    out_type=jax.ShapeDtypeStruct((NC, N2, 128), jnp.float32),
    mesh=_mesh,
    scratch_types=[
        pltpu.VMEM((CHUNK,), jnp.int32),
        pltpu.VMEM((CHUNK,), jnp.int32),
        pltpu.VMEM((CHUNK, 128), jnp.float32),
        pltpu.VMEM_SHARED((N2, 128), jnp.float32),
    ],
)

_agg2 = pl.kernel(
    _sc_agg2,
    out_type=jax.ShapeDtypeStruct((NC, N2), jnp.float32),
    mesh=_mesh,
    scratch_types=[
        pltpu.VMEM((CHUNK,), jnp.int32),
        pltpu.VMEM((CHUNK,), jnp.int32),
        pltpu.VMEM((CHUNK,), jnp.float32),
        pltpu.VMEM_SHARED((N2,), jnp.float32),
    ],
)

_BN_S = 1.0 / (1.0 + EPS) ** 0.5
_RB = 1024


def _tc_layer0(aggp, cntp, xp, wl0, wr0, b0, g0, be0, out):
    i = pl.program_id(0)
    agg = aggp[0] + aggp[1]
    cnt = cntp[0, pl.ds(i * _RB, _RB)] + cntp[1, pl.ds(i * _RB, _RB)]
    inv = 1.0 / jnp.maximum(cnt, 1.0)
    mean = agg * inv[:, None]
    z = (lax.dot_general(mean, wl0[...], (((1,), (1,)), ((), ())),
                         preferred_element_type=jnp.float32)
         + lax.dot_general(xp[...], wr0[...], (((1,), (1,)), ((), ())),
                           preferred_element_type=jnp.float32)
         + b0[0])
    h = z * (g0[0] * _BN_S) + be0[0]
    out[...] = jnp.maximum(h, 0.0)[None]


def _tc_layer1(aggp, cntp, h1p, wl1, wr1, b1, g1, be1, w2, out):
    i = pl.program_id(0)
    a = jnp.concatenate([aggp[0], aggp[1]], axis=1)
    hv = jnp.concatenate([h1p[0], h1p[1]], axis=1)
    cnt = cntp[0, pl.ds(i * _RB, _RB)] + cntp[1, pl.ds(i * _RB, _RB)]
    inv = 1.0 / jnp.maximum(cnt, 1.0)
    mean = a * inv[:, None]
    z = (lax.dot_general(mean, wl1[...], (((1,), (1,)), ((), ())),
                         preferred_element_type=jnp.float32)
         + lax.dot_general(hv, wr1[...], (((1,), (1,)), ((), ())),
                           preferred_element_type=jnp.float32)
         + b1[...])
    h2 = jnp.maximum(z * (g1[...] * _BN_S) + be1[...], 0.0)
    out[...] = jnp.dot(h2, w2[...], preferred_element_type=jnp.float32)


def _tc_final(tp, cntp, r2, b2, out):
    t = tp[...]
    tagg = t[0:1, :] + t[1:2, :]
    cnt = cntp[0:1, :] + cntp[1:2, :]
    inv = 1.0 / jnp.maximum(cnt, 1.0)
    val = tagg * inv + r2[...] + b2[0, 0]
    out[...] = jax.nn.sigmoid(val)


def kernel(x, edge_index, W_l0, b0, W_r0, gamma0, beta0,
           W_l1, b1, W_r1, gamma1, beta1, W_l2, b2, W_r2):
    f32 = jnp.float32
    src = edge_index[0]
    dst = edge_index[1]
    pad = E_PAD - E
    src_p = jnp.concatenate([src, jnp.zeros((pad,), jnp.int32)])
    dst_p = jnp.concatenate([dst, jnp.full((pad,), N, jnp.int32)])
    idx1 = jnp.concatenate([src_p, src_p + N2])
    xp = jnp.pad(x, ((0, N2 - N), (0, 0)))
    zrows = jnp.zeros((N2, 128), f32)
    zvec = jnp.zeros((N2,), f32)
    b0r = b0.reshape(2, 1, 128)
    g0r = gamma0.reshape(2, 1, 128)
    be0r = beta0.reshape(2, 1, 128)
    b1r = b1.reshape(1, D_H)
    g1r = gamma1.reshape(1, D_H)
    be1r = beta1.reshape(1, D_H)
    w2cat = jnp.concatenate([W_l2, W_r2], axis=0).T
    b2r = b2.reshape(1, 1)

    agg0p, cntp = _agg0(xp, src_p, dst_p, zrows, zvec)

    nblk = N2 // _RB
    h1s = pl.pallas_call(
        _tc_layer0,
        grid=(nblk, 2),
        in_specs=[
            pl.BlockSpec((NC, _RB, 128), lambda i, c: (0, i, 0)),
            pl.BlockSpec((NC, N2), lambda i, c: (0, 0)),
            pl.BlockSpec((_RB, 128), lambda i, c: (i, 0)),
            pl.BlockSpec((128, 128), lambda i, c: (c, 0)),
            pl.BlockSpec((128, 128), lambda i, c: (c, 0)),
            pl.BlockSpec((1, 1, 128), lambda i, c: (c, 0, 0)),
            pl.BlockSpec((1, 1, 128), lambda i, c: (c, 0, 0)),
            pl.BlockSpec((1, 1, 128), lambda i, c: (c, 0, 0)),
        ],
        out_specs=pl.BlockSpec((1, _RB, 128), lambda i, c: (c, i, 0)),
        out_shape=jax.ShapeDtypeStruct((2, N2, 128), f32),
    )(agg0p, cntp, xp, W_l0, W_r0, b0r, g0r, be0r)

    h1flat = h1s.reshape(2 * N2, 128)
    agg1p = _agg1(h1flat, idx1, dst_p, zrows)

    tr = pl.pallas_call(
        _tc_layer1,
        grid=(nblk,),
        in_specs=[
            pl.BlockSpec((NC, _RB, 128), lambda i: (0, i, 0)),
            pl.BlockSpec((NC, N2), lambda i: (0, 0)),
            pl.BlockSpec((NC, _RB, 128), lambda i: (0, i, 0)),
            pl.BlockSpec((D_H, D_H), lambda i: (0, 0)),
            pl.BlockSpec((D_H, D_H), lambda i: (0, 0)),
            pl.BlockSpec((1, D_H), lambda i: (0, 0)),
            pl.BlockSpec((1, D_H), lambda i: (0, 0)),
            pl.BlockSpec((1, D_H), lambda i: (0, 0)),
            pl.BlockSpec((D_H, 2), lambda i: (0, 0)),
        ],
        out_specs=pl.BlockSpec((_RB, 2), lambda i: (i, 0)),
        out_shape=jax.ShapeDtypeStruct((N2, 2), f32),
    )(agg1p, cntp, h1s, W_l1, W_r1, b1r, g1r, be1r, w2cat)

    t2 = tr[:, 0]
    r2row = tr[:, 1].reshape(1, N2)
    t2p = _agg2(t2, src_p, dst_p, zvec)

    outrow = pl.pallas_call(
        _tc_final,
        in_specs=[
            pl.BlockSpec((NC, N2), lambda: (0, 0)),
            pl.BlockSpec((NC, N2), lambda: (0, 0)),
            pl.BlockSpec((1, N2), lambda: (0, 0)),
            pl.BlockSpec((1, 1), lambda: (0, 0)),
        ],
        out_specs=pl.BlockSpec((1, N2), lambda: (0, 0)),
        out_shape=jax.ShapeDtypeStruct((1, N2), f32),
    )(t2p, cntp, r2row, b2r)

    return outrow[0, :N].reshape(N, 1)


# speedup vs baseline: 5.3756x; 1.0048x over previous
import jax
import jax.numpy as jnp
from jax import lax
from jax.experimental import pallas as pl
from jax.experimental.pallas import tpu as pltpu
from jax.experimental.pallas import tpu_sc as plsc

N = 10000
E = 320000
D_IN = 128
D_H = 256
EPS = 1e-5

N2 = 10240
E_PAD = 327680
NC = 2
NS = 16
CHUNK = 128
RPT = N2 // NS
NROW = E_PAD // CHUNK
STRIP = 8

_mesh = plsc.VectorSubcoreMesh(core_axis_name="c", subcore_axis_name="s")


def _strip_pipeline(nch, table, src_hbm, dst_hbm, idx_base, dst_base,
                    sbs, dbs, rows, acc_sh, gsem, ssem, isem, extra=None):
    nstrips = nch // STRIP

    def idx_issue(g, p):
        pltpu.async_copy(src_hbm.at[pl.ds(idx_base + g * STRIP, STRIP)],
                         sbs[p], isem.at[p])
        pltpu.async_copy(dst_hbm.at[pl.ds(dst_base + g * STRIP, STRIP)],
                         dbs[p], isem.at[p])

    def idx_wait(p):
        pltpu.make_async_copy(src_hbm.at[pl.ds(idx_base, STRIP)],
                              sbs[p], isem.at[p]).wait()
        pltpu.make_async_copy(dst_hbm.at[pl.ds(dst_base, STRIP)],
                              dbs[p], isem.at[p]).wait()

    def gather(p, jj, b):
        pltpu.async_copy(table.at[sbs[p].at[jj]], rows[b], gsem.at[b])

    def wait_gather(b):
        pltpu.make_async_copy(table.at[sbs[0].at[0]], rows[b],
                              gsem.at[b]).wait()

    def scatter(p, jj, b):
        pltpu.async_copy(rows[b], acc_sh.at[dbs[p].at[jj]], ssem.at[b],
                         add=True)

    def wait_scatter(b):
        pltpu.make_async_copy(rows[b], acc_sh.at[dbs[0].at[0]],
                              ssem.at[b]).wait()

    idx_issue(0, 0)

    def pair(g2, carry):
        for sp in range(2):
            g = 2 * g2 + sp
            idx_wait(sp)

            @pl.when(g + 1 < nstrips)
            def _():
                idx_issue(g + 1, 1 - sp)

            @pl.when(g > 0)
            def _():
                wait_scatter(0)

            gather(sp, 0, 0)
            for jj in range(STRIP):
                j = g * STRIP + jj
                b = jj % 2
                wait_gather(b)
                scatter(sp, jj, b)
                if extra is not None:
                    extra(sp, jj, j)
                if jj < STRIP - 1:
                    @pl.when(j >= 1)
                    def _():
                        wait_scatter(1 - b)

                    gather(sp, jj + 1, 1 - b)
        return carry

    lax.fori_loop(0, nstrips // 2, pair, 0)
    wait_scatter(0)
    wait_scatter(1)


def _sc_agg0(x_hbm, src_hbm, dst_hbm, zrows_hbm, zvec_hbm,
             agg_out, cnt_out,
             sb0, sb1, db0, db1, rr0, rr1, ones_v,
             acc_sh, cnt_sh, gsem, ssem, isem, csem):
    c = lax.axis_index("c")
    s = lax.axis_index("s")
    pltpu.sync_copy(zrows_hbm.at[pl.ds(s * RPT, RPT)],
                    acc_sh.at[pl.ds(s * RPT, RPT)])
    pltpu.sync_copy(zvec_hbm.at[pl.ds(s * RPT, RPT)],
                    cnt_sh.at[pl.ds(s * RPT, RPT)])
    for i in range(8):
        ones_v[pl.ds(i * 16, 16)] = jnp.full((16,), 1.0, jnp.float32)
    plsc.subcore_barrier()

    nch = E_PAD // (NC * NS) // CHUNK
    rb = c * (NROW // NC) + s * nch

    def count(sp, jj, j):
        db = (db0, db1)[sp]
        pltpu.async_copy(ones_v, cnt_sh.at[db.at[jj]], csem, add=True)

        @pl.when(j > 0)
        def _():
            pltpu.make_async_copy(ones_v, cnt_sh.at[db0.at[0]], csem).wait()

    _strip_pipeline(nch, x_hbm, src_hbm, dst_hbm, rb, rb,
                    (sb0, sb1), (db0, db1), (rr0, rr1),
                    acc_sh, gsem, ssem, isem, extra=count)
    pltpu.make_async_copy(ones_v, cnt_sh.at[db0.at[0]], csem).wait()
    plsc.subcore_barrier()
    pltpu.sync_copy(acc_sh.at[pl.ds(s * RPT, RPT)],
                    agg_out.at[c, pl.ds(s * RPT, RPT)])
    pltpu.sync_copy(cnt_sh.at[pl.ds(s * RPT, RPT)],
                    cnt_out.at[c, pl.ds(s * RPT, RPT)])


def _sc_agg1(h_hbm, idx_hbm, dst_hbm, zrows_hbm,
             agg_out,
             sb0, sb1, db0, db1, rr0, rr1, acc_sh, gsem, ssem, isem):
    c = lax.axis_index("c")
    s = lax.axis_index("s")
    pltpu.sync_copy(zrows_hbm.at[pl.ds(s * RPT, RPT)],
                    acc_sh.at[pl.ds(s * RPT, RPT)])
    plsc.subcore_barrier()

    nch = E_PAD // NS // CHUNK
    _strip_pipeline(nch, h_hbm, idx_hbm, dst_hbm,
                    c * NROW + s * nch, s * nch,
                    (sb0, sb1), (db0, db1), (rr0, rr1),
                    acc_sh, gsem, ssem, isem)
    plsc.subcore_barrier()
    pltpu.sync_copy(acc_sh.at[pl.ds(s * RPT, RPT)],
                    agg_out.at[c, pl.ds(s * RPT, RPT)])


def _sc_agg2(t_hbm, src_hbm, dst_hbm, zvec_hbm,
             agg_out,
             sb0, sb1, db0, db1, rr0, rr1, acc_sh, gsem, ssem, isem):
    c = lax.axis_index("c")
    s = lax.axis_index("s")
    pltpu.sync_copy(zvec_hbm.at[pl.ds(s * RPT, RPT)],
                    acc_sh.at[pl.ds(s * RPT, RPT)])
    plsc.subcore_barrier()

    nch = E_PAD // (NC * NS) // CHUNK
    rb = c * (NROW // NC) + s * nch
    _strip_pipeline(nch, t_hbm, src_hbm, dst_hbm, rb, rb,
                    (sb0, sb1), (db0, db1), (rr0, rr1),
                    acc_sh, gsem, ssem, isem)
    plsc.subcore_barrier()
    pltpu.sync_copy(acc_sh.at[pl.ds(s * RPT, RPT)],
                    agg_out.at[c, pl.ds(s * RPT, RPT)])


_agg0 = pl.kernel(
    _sc_agg0,
    out_type=(jax.ShapeDtypeStruct((NC, N2, D_IN), jnp.float32),
              jax.ShapeDtypeStruct((NC, N2), jnp.float32)),
    mesh=_mesh,
    scratch_types=[
        pltpu.VMEM((STRIP, CHUNK), jnp.int32),
        pltpu.VMEM((STRIP, CHUNK), jnp.int32),
        pltpu.VMEM((STRIP, CHUNK), jnp.int32),
        pltpu.VMEM((STRIP, CHUNK), jnp.int32),
        pltpu.VMEM((CHUNK, D_IN), jnp.float32),
        pltpu.VMEM((CHUNK, D_IN), jnp.float32),
        pltpu.VMEM((CHUNK,), jnp.float32),
        pltpu.VMEM_SHARED((N2, D_IN), jnp.float32),
        pltpu.VMEM_SHARED((N2,), jnp.float32),
        pltpu.SemaphoreType.DMA((2,)),
        pltpu.SemaphoreType.DMA((2,)),
        pltpu.SemaphoreType.DMA((2,)),
        pltpu.SemaphoreType.DMA,
    ],
)

_agg1 = pl.kernel(
    _sc_agg1,
    out_type=jax.ShapeDtypeStruct((NC, N2, 128), jnp.float32),
    mesh=_mesh,
    scratch_types=[
        pltpu.VMEM((STRIP, CHUNK), jnp.int32),
        pltpu.VMEM((STRIP, CHUNK), jnp.int32),
        pltpu.VMEM((STRIP, CHUNK), jnp.int32),
        pltpu.VMEM((STRIP, CHUNK), jnp.int32),
        pltpu.VMEM((CHUNK, 128), jnp.float32),
        pltpu.VMEM((CHUNK, 128), jnp.float32),
        pltpu.VMEM_SHARED((N2, 128), jnp.float32),
        pltpu.SemaphoreType.DMA((2,)),
        pltpu.SemaphoreType.DMA((2,)),
        pltpu.SemaphoreType.DMA((2,)),
    ],
)

_agg2 = pl.kernel(
    _sc_agg2,
    out_type=jax.ShapeDtypeStruct((NC, N2), jnp.float32),
    mesh=_mesh,
    scratch_types=[
        pltpu.VMEM((STRIP, CHUNK), jnp.int32),
        pltpu.VMEM((STRIP, CHUNK), jnp.int32),
        pltpu.VMEM((STRIP, CHUNK), jnp.int32),
        pltpu.VMEM((STRIP, CHUNK), jnp.int32),
        pltpu.VMEM((CHUNK,), jnp.float32),
        pltpu.VMEM((CHUNK,), jnp.float32),
        pltpu.VMEM_SHARED((N2,), jnp.float32),
        pltpu.SemaphoreType.DMA((2,)),
        pltpu.SemaphoreType.DMA((2,)),
        pltpu.SemaphoreType.DMA((2,)),
    ],
)

_BN_S = 1.0 / (1.0 + EPS) ** 0.5
_RB = 1024


def _tc_layer0(aggp, cntp, xp, wl0, wr0, b0, g0, be0, out):
    i = pl.program_id(0)
    agg = aggp[0] + aggp[1]
    cnt = cntp[0, pl.ds(i * _RB, _RB)] + cntp[1, pl.ds(i * _RB, _RB)]
    inv = 1.0 / jnp.maximum(cnt, 1.0)
    mean = agg * inv[:, None]
    z = (lax.dot_general(mean, wl0[...], (((1,), (1,)), ((), ())),
                         preferred_element_type=jnp.float32)
         + lax.dot_general(xp[...], wr0[...], (((1,), (1,)), ((), ())),
                           preferred_element_type=jnp.float32)
         + b0[0])
    h = z * (g0[0] * _BN_S) + be0[0]
    out[...] = jnp.maximum(h, 0.0)[None]


def _tc_layer1(aggp, cntp, h1p, wl1, wr1, b1, g1, be1, w2, out):
    i = pl.program_id(0)
    a = jnp.concatenate([aggp[0], aggp[1]], axis=1)
    hv = jnp.concatenate([h1p[0], h1p[1]], axis=1)
    cnt = cntp[0, pl.ds(i * _RB, _RB)] + cntp[1, pl.ds(i * _RB, _RB)]
    inv = 1.0 / jnp.maximum(cnt, 1.0)
    mean = a * inv[:, None]
    z = (lax.dot_general(mean, wl1[...], (((1,), (1,)), ((), ())),
                         preferred_element_type=jnp.float32)
         + lax.dot_general(hv, wr1[...], (((1,), (1,)), ((), ())),
                           preferred_element_type=jnp.float32)
         + b1[...])
    h2 = jnp.maximum(z * (g1[...] * _BN_S) + be1[...], 0.0)
    out[...] = jnp.dot(h2, w2[...], preferred_element_type=jnp.float32)


def _tc_final(tp, cntp, r2, b2, out):
    t = tp[...]
    tagg = t[0:1, :] + t[1:2, :]
    cnt = cntp[0:1, :] + cntp[1:2, :]
    inv = 1.0 / jnp.maximum(cnt, 1.0)
    val = tagg * inv + r2[...] + b2[0, 0]
    out[...] = jax.nn.sigmoid(val)


def kernel(x, edge_index, W_l0, b0, W_r0, gamma0, beta0,
           W_l1, b1, W_r1, gamma1, beta1, W_l2, b2, W_r2):
    f32 = jnp.float32
    src = edge_index[0]
    dst = edge_index[1]
    pad = E_PAD - E
    src_p = jnp.concatenate([src, jnp.zeros((pad,), jnp.int32)])
    dst_p = jnp.concatenate([dst, jnp.full((pad,), N, jnp.int32)])
    idx1 = jnp.concatenate([src_p, src_p + N2]).reshape(2 * NROW, CHUNK)
    src2d = src_p.reshape(NROW, CHUNK)
    dst2d = dst_p.reshape(NROW, CHUNK)
    xp = jnp.pad(x, ((0, N2 - N), (0, 0)))
    zrows = jnp.zeros((N2, 128), f32)
    zvec = jnp.zeros((N2,), f32)
    b0r = b0.reshape(2, 1, 128)
    g0r = gamma0.reshape(2, 1, 128)
    be0r = beta0.reshape(2, 1, 128)
    b1r = b1.reshape(1, D_H)
    g1r = gamma1.reshape(1, D_H)
    be1r = beta1.reshape(1, D_H)
    w2cat = jnp.concatenate([W_l2, W_r2], axis=0).T
    b2r = b2.reshape(1, 1)

    agg0p, cntp = _agg0(xp, src2d, dst2d, zrows, zvec)

    nblk = N2 // _RB
    h1s = pl.pallas_call(
        _tc_layer0,
        grid=(nblk, 2),
        in_specs=[
            pl.BlockSpec((NC, _RB, 128), lambda i, c: (0, i, 0)),
            pl.BlockSpec((NC, N2), lambda i, c: (0, 0)),
            pl.BlockSpec((_RB, 128), lambda i, c: (i, 0)),
            pl.BlockSpec((128, 128), lambda i, c: (c, 0)),
            pl.BlockSpec((128, 128), lambda i, c: (c, 0)),
            pl.BlockSpec((1, 1, 128), lambda i, c: (c, 0, 0)),
            pl.BlockSpec((1, 1, 128), lambda i, c: (c, 0, 0)),
            pl.BlockSpec((1, 1, 128), lambda i, c: (c, 0, 0)),
        ],
        out_specs=pl.BlockSpec((1, _RB, 128), lambda i, c: (c, i, 0)),
        out_shape=jax.ShapeDtypeStruct((2, N2, 128), f32),
    )(agg0p, cntp, xp, W_l0, W_r0, b0r, g0r, be0r)

    h1flat = h1s.reshape(2 * N2, 128)
    agg1p = _agg1(h1flat, idx1, dst2d, zrows)

    tr = pl.pallas_call(
        _tc_layer1,
        grid=(nblk,),
        in_specs=[
            pl.BlockSpec((NC, _RB, 128), lambda i: (0, i, 0)),
            pl.BlockSpec((NC, N2), lambda i: (0, 0)),
            pl.BlockSpec((NC, _RB, 128), lambda i: (0, i, 0)),
            pl.BlockSpec((D_H, D_H), lambda i: (0, 0)),
            pl.BlockSpec((D_H, D_H), lambda i: (0, 0)),
            pl.BlockSpec((1, D_H), lambda i: (0, 0)),
            pl.BlockSpec((1, D_H), lambda i: (0, 0)),
            pl.BlockSpec((1, D_H), lambda i: (0, 0)),
            pl.BlockSpec((D_H, 2), lambda i: (0, 0)),
        ],
        out_specs=pl.BlockSpec((_RB, 2), lambda i: (i, 0)),
        out_shape=jax.ShapeDtypeStruct((N2, 2), f32),
    )(agg1p, cntp, h1s, W_l1, W_r1, b1r, g1r, be1r, w2cat)

    t2 = tr[:, 0]
    r2row = tr[:, 1].reshape(1, N2)
    t2p = _agg2(t2, src2d, dst2d, zvec)

    outrow = pl.pallas_call(
        _tc_final,
        in_specs=[
            pl.BlockSpec((NC, N2), lambda: (0, 0)),
            pl.BlockSpec((NC, N2), lambda: (0, 0)),
            pl.BlockSpec((1, N2), lambda: (0, 0)),
            pl.BlockSpec((1, 1), lambda: (0, 0)),
        ],
        out_specs=pl.BlockSpec((1, N2), lambda: (0, 0)),
        out_shape=jax.ShapeDtypeStruct((1, N2), f32),
    )(t2p, cntp, r2row, b2r)

    return outrow[0, :N].reshape(N, 1)
